# Initial kernel scaffold; baseline (speedup 1.0000x reference)
import jax
import jax.numpy as jnp
from jax.experimental import pallas as pl


def kernel(x, edge_index, batch, W1, b1, W2, b2, W3, b3, W4, b4, Wl, bl):
    raise NotImplementedError("write your pallas kernel here")


# speedup vs baseline: 16.4258x; 16.4258x over previous
import functools

import jax
import jax.numpy as jnp
from jax import lax
from jax.experimental import pallas as pl
from jax.experimental.pallas import tpu as pltpu
from jax.experimental.pallas import tpu_sc as plsc

_N = 10000
_E = 320000
_IN = 128
_HID = 64
_OUT = 128
_G = 64
_W = 128

_NC = 2
_NS = 16
_NW = _NC * _NS
_C = 128
_K = 80
_EPAD = _NW * _K * _C
_NPAD = 10112
_RPT = _NPAD // _NS

_mesh = plsc.VectorSubcoreMesh(core_axis_name="c", subcore_axis_name="s")


@functools.partial(
    pl.kernel,
    out_type=jax.ShapeDtypeStruct((_NC, _NPAD, _W), jnp.float32),
    mesh=_mesh,
    scratch_types=[
        pltpu.VMEM((_K, _C), jnp.int32),
        pltpu.VMEM((_C, _W), jnp.float32),
        pltpu.VMEM_SHARED((_NPAD, _W), jnp.float32),
    ],
)
def _sc_degree(dstb, zeros, ones_hbm, out, didx, ones_v, acc):
    c = lax.axis_index("c")
    s = lax.axis_index("s")
    w = c * _NS + s
    r0 = s * _RPT

    pltpu.sync_copy(zeros.at[pl.ds(r0, _RPT)], acc.at[pl.ds(r0, _RPT)])
    pltpu.sync_copy(dstb.at[w], didx)
    pltpu.sync_copy(ones_hbm, ones_v)
    plsc.subcore_barrier()

    def step(k, carry):
        pltpu.sync_copy(ones_v, acc.at[didx.at[k]], add=True)
        return carry

    lax.fori_loop(0, _K, step, 0)
    plsc.subcore_barrier()
    pltpu.sync_copy(acc.at[pl.ds(r0, _RPT)], out.at[c, pl.ds(r0, _RPT)])


@functools.partial(
    pl.kernel,
    out_type=jax.ShapeDtypeStruct((_NC, _NPAD, _W), jnp.float32),
    mesh=_mesh,
    scratch_types=[
        pltpu.VMEM((_K, _C), jnp.int32),
        pltpu.VMEM((_K, _C), jnp.int32),
        pltpu.VMEM((_C, _W), jnp.float32),
        pltpu.SemaphoreType.DMA,
        pltpu.VMEM_SHARED((_NPAD, _W), jnp.float32),
    ],
)
def _sc_aggregate(hs, srcb, dstb, zeros, out, sidx, didx, rows, gsem, acc):
    c = lax.axis_index("c")
    s = lax.axis_index("s")
    w = c * _NS + s
    r0 = s * _RPT

    pltpu.sync_copy(zeros.at[pl.ds(r0, _RPT)], acc.at[pl.ds(r0, _RPT)])
    pltpu.sync_copy(srcb.at[w], sidx)
    pltpu.sync_copy(dstb.at[w], didx)
    plsc.subcore_barrier()

    def step(k, carry):
        pltpu.async_copy(hs.at[sidx.at[k]], rows, gsem).wait()
        pltpu.sync_copy(rows, acc.at[didx.at[k]], add=True)
        return carry

    lax.fori_loop(0, _K, step, 0)
    plsc.subcore_barrier()
    pltpu.sync_copy(acc.at[pl.ds(r0, _RPT)], out.at[c, pl.ds(r0, _RPT)])


def _tc_prep_body(x_ref, w_ref, dp_ref, hs_ref, dis_ref):
    deg = dp_ref[0, : _N, 0:1] + dp_ref[1, : _N, 0:1] + 1.0
    dis = lax.rsqrt(deg)
    h = jnp.dot(x_ref[...], w_ref[...], preferred_element_type=jnp.float32)
    hs_ref[0:_N, 0:_HID] = h * dis
    hs_ref[0:_N, _HID:_W] = jnp.zeros((_N, _W - _HID), jnp.float32)
    hs_ref[_N:_NPAD, :] = jnp.zeros((_NPAD - _N, _W), jnp.float32)
    dis_ref[...] = dis


def _tc_mid_body(p_ref, hs_ref, dis_ref, b_ref, w_ref, o_ref):
    dis = dis_ref[...]
    agg = (p_ref[0, : _N, 0:_HID] + p_ref[1, : _N, 0:_HID]
           + hs_ref[0:_N, 0:_HID]) * dis
    a = jnp.maximum(agg + b_ref[...], 0.0)
    o_ref[0:_N, 0:_HID] = (
        jnp.dot(a, w_ref[...], preferred_element_type=jnp.float32) * dis
    )
    o_ref[0:_N, _HID:_W] = jnp.zeros((_N, _W - _HID), jnp.float32)
    o_ref[_N:_NPAD, :] = jnp.zeros((_NPAD - _N, _W), jnp.float32)


def _tc_final_body(p_ref, hs_ref, dis_ref, b_ref, bt_ref, wl_ref, bl_ref, o_ref):
    dis = dis_ref[...]
    agg = (p_ref[0, : _N, 0:_HID] + p_ref[1, : _N, 0:_HID]
           + hs_ref[0:_N, 0:_HID]) * dis
    a = jnp.maximum(agg + b_ref[...], 0.0)
    gids = lax.broadcasted_iota(jnp.int32, (_G, _N), 0)
    oh = (bt_ref[...] == gids).astype(jnp.float32)
    sums = jnp.dot(oh, a, preferred_element_type=jnp.float32)
    cnt = jnp.sum(oh, axis=1, keepdims=True)
    pooled = sums / jnp.maximum(cnt, 1.0)
    o_ref[...] = (
        jnp.dot(pooled, wl_ref[...], preferred_element_type=jnp.float32)
        + bl_ref[...]
    )


_tc_prep = pl.pallas_call(
    _tc_prep_body,
    out_shape=(
        jax.ShapeDtypeStruct((_NPAD, _W), jnp.float32),
        jax.ShapeDtypeStruct((_N, 1), jnp.float32),
    ),
)

_tc_mid = pl.pallas_call(
    _tc_mid_body,
    out_shape=jax.ShapeDtypeStruct((_NPAD, _W), jnp.float32),
)

_tc_final = pl.pallas_call(
    _tc_final_body,
    out_shape=jax.ShapeDtypeStruct((_G, _OUT), jnp.float32),
)


def kernel(x, edge_index, batch, W1, b1, W2, b2, W3, b3, W4, b4, Wl, bl):
    src = edge_index[0]
    dst = edge_index[1]
    npad = _EPAD - _E
    ar = jnp.arange(npad, dtype=jnp.int32)
    srcb = jnp.concatenate([src, ar % 128]).reshape(_NW, _K, _C)
    dstb = jnp.concatenate([dst, _N + (ar % (_NPAD - _N))]).reshape(_NW, _K, _C)

    zeros = jnp.zeros((_NPAD, _W), jnp.float32)
    ones = jnp.ones((_C, _W), jnp.float32)

    dp = _sc_degree(dstb, zeros, ones)
    hs, dis = _tc_prep(x, W1, dp)

    for b_prev, W_next in ((b1, W2), (b2, W3), (b3, W4)):
        p = _sc_aggregate(hs, srcb, dstb, zeros)
        hs = _tc_mid(p, hs, dis, b_prev.reshape(1, _HID), W_next)

    p = _sc_aggregate(hs, srcb, dstb, zeros)
    return _tc_final(
        p, hs, dis, b4.reshape(1, _HID), batch.reshape(1, _N), Wl,
        bl.reshape(1, _OUT),
    )


# speedup vs baseline: 24.2198x; 1.4745x over previous
import functools

import jax
import jax.numpy as jnp
from jax import lax
from jax.experimental import pallas as pl
from jax.experimental.pallas import tpu as pltpu
from jax.experimental.pallas import tpu_sc as plsc

_N = 10000
_E = 320000
_IN = 128
_HID = 64
_OUT = 128
_G = 64
_W = 128

_NC = 2
_NS = 16
_NW = _NC * _NS
_C = 128
_K = 80
_EPAD = _NW * _K * _C
_NPAD = 10112
_RPT = _NPAD // _NS

_mesh = plsc.VectorSubcoreMesh(core_axis_name="c", subcore_axis_name="s")


@functools.partial(
    pl.kernel,
    out_type=jax.ShapeDtypeStruct((_NC, _NPAD, _W), jnp.float32),
    mesh=_mesh,
    scratch_types=[
        pltpu.VMEM((_K, _C), jnp.int32),
        pltpu.VMEM((_C, _W), jnp.float32),
        pltpu.SemaphoreType.DMA,
        pltpu.VMEM_SHARED((_NPAD, _W), jnp.float32),
    ],
)
def _sc_degree(dstb, zeros, ones_hbm, out, didx, ones_v, ssem, acc):
    c = lax.axis_index("c")
    s = lax.axis_index("s")
    w = c * _NS + s
    r0 = s * _RPT

    pltpu.sync_copy(zeros.at[pl.ds(r0, _RPT)], acc.at[pl.ds(r0, _RPT)])
    pltpu.sync_copy(dstb.at[w], didx)
    pltpu.sync_copy(ones_hbm, ones_v)
    plsc.subcore_barrier()

    def group(gi, carry):
        base = gi * 8
        descs = [
            pltpu.async_copy(ones_v, acc.at[didx.at[base + j]], ssem, add=True)
            for j in range(8)
        ]
        for d in descs:
            d.wait()
        return carry

    lax.fori_loop(0, _K // 8, group, 0)
    plsc.subcore_barrier()
    pltpu.sync_copy(acc.at[pl.ds(r0, _RPT)], out.at[c, pl.ds(r0, _RPT)])


@functools.partial(
    pl.kernel,
    out_type=jax.ShapeDtypeStruct((_NC, _NPAD, _W), jnp.float32),
    mesh=_mesh,
    scratch_types=[
        pltpu.VMEM((_K, _C), jnp.int32),
        pltpu.VMEM((2, _C), jnp.int32),
        pltpu.VMEM((2, _C, _W), jnp.float32),
        pltpu.SemaphoreType.DMA,
        pltpu.SemaphoreType.DMA,
        pltpu.SemaphoreType.DMA,
        pltpu.SemaphoreType.DMA,
        pltpu.SemaphoreType.DMA,
        pltpu.SemaphoreType.DMA,
        pltpu.VMEM_SHARED((_NPAD, _W), jnp.float32),
    ],
)
def _sc_aggregate(hs, srcb, dstb, zeros, out, sidx, didx, rows, gsem0, gsem1,
                  ssem0, ssem1, isem0, isem1, acc):
    c = lax.axis_index("c")
    s = lax.axis_index("s")
    w = c * _NS + s
    r0 = s * _RPT

    pltpu.sync_copy(zeros.at[pl.ds(r0, _RPT)], acc.at[pl.ds(r0, _RPT)])
    pltpu.sync_copy(srcb.at[w], sidx)
    plsc.subcore_barrier()

    def g_start(k, buf, gsem, isem):
        pltpu.async_copy(dstb.at[w, k], didx.at[buf], isem)
        pltpu.async_copy(hs.at[sidx.at[k]], rows.at[buf], gsem)

    def g_wait(buf, gsem, isem):
        pltpu.make_async_copy(hs.at[sidx.at[0]], rows.at[buf], gsem).wait()
        pltpu.make_async_copy(dstb.at[w, 0], didx.at[buf], isem).wait()

    def s_start(buf, ssem):
        pltpu.async_copy(rows.at[buf], acc.at[didx.at[buf]], ssem, add=True)

    def s_wait(buf, ssem):
        pltpu.make_async_copy(rows.at[buf], acc.at[didx.at[buf]], ssem).wait()

    g_start(0, 0, gsem0, isem0)

    def step(i, carry):
        k0 = 2 * i
        pl.when(i >= 1)(lambda: s_wait(1, ssem1))
        g_start(k0 + 1, 1, gsem1, isem1)
        g_wait(0, gsem0, isem0)
        s_start(0, ssem0)

        @pl.when(i + 1 < _K // 2)
        def _():
            s_wait(0, ssem0)
            g_start(k0 + 2, 0, gsem0, isem0)

        g_wait(1, gsem1, isem1)
        s_start(1, ssem1)
        return carry

    lax.fori_loop(0, _K // 2, step, 0)
    s_wait(0, ssem0)
    s_wait(1, ssem1)
    plsc.subcore_barrier()
    pltpu.sync_copy(acc.at[pl.ds(r0, _RPT)], out.at[c, pl.ds(r0, _RPT)])


def _tc_prep_body(x_ref, w_ref, dp_ref, hs_ref, dis_ref):
    deg = dp_ref[0, : _N, 0:1] + dp_ref[1, : _N, 0:1] + 1.0
    dis = lax.rsqrt(deg)
    h = jnp.dot(x_ref[...], w_ref[...], preferred_element_type=jnp.float32)
    hs_ref[0:_N, 0:_HID] = h * dis
    hs_ref[0:_N, _HID:_W] = jnp.zeros((_N, _W - _HID), jnp.float32)
    hs_ref[_N:_NPAD, :] = jnp.zeros((_NPAD - _N, _W), jnp.float32)
    dis_ref[...] = dis


def _tc_mid_body(p_ref, hs_ref, dis_ref, b_ref, w_ref, o_ref):
    dis = dis_ref[...]
    agg = (p_ref[0, : _N, 0:_HID] + p_ref[1, : _N, 0:_HID]
           + hs_ref[0:_N, 0:_HID]) * dis
    a = jnp.maximum(agg + b_ref[...], 0.0)
    o_ref[0:_N, 0:_HID] = (
        jnp.dot(a, w_ref[...], preferred_element_type=jnp.float32) * dis
    )
    o_ref[0:_N, _HID:_W] = jnp.zeros((_N, _W - _HID), jnp.float32)
    o_ref[_N:_NPAD, :] = jnp.zeros((_NPAD - _N, _W), jnp.float32)


def _tc_final_body(p_ref, hs_ref, dis_ref, b_ref, bt_ref, wl_ref, bl_ref, o_ref):
    dis = dis_ref[...]
    agg = (p_ref[0, : _N, 0:_HID] + p_ref[1, : _N, 0:_HID]
           + hs_ref[0:_N, 0:_HID]) * dis
    a = jnp.maximum(agg + b_ref[...], 0.0)
    gids = lax.broadcasted_iota(jnp.int32, (_G, _N), 0)
    oh = (bt_ref[...] == gids).astype(jnp.float32)
    sums = jnp.dot(oh, a, preferred_element_type=jnp.float32)
    cnt = jnp.sum(oh, axis=1, keepdims=True)
    pooled = sums / jnp.maximum(cnt, 1.0)
    o_ref[...] = (
        jnp.dot(pooled, wl_ref[...], preferred_element_type=jnp.float32)
        + bl_ref[...]
    )


_tc_prep = pl.pallas_call(
    _tc_prep_body,
    out_shape=(
        jax.ShapeDtypeStruct((_NPAD, _W), jnp.float32),
        jax.ShapeDtypeStruct((_N, 1), jnp.float32),
    ),
)

_tc_mid = pl.pallas_call(
    _tc_mid_body,
    out_shape=jax.ShapeDtypeStruct((_NPAD, _W), jnp.float32),
)

_tc_final = pl.pallas_call(
    _tc_final_body,
    out_shape=jax.ShapeDtypeStruct((_G, _OUT), jnp.float32),
)


def kernel(x, edge_index, batch, W1, b1, W2, b2, W3, b3, W4, b4, Wl, bl):
    src = edge_index[0]
    dst = edge_index[1]
    npad = _EPAD - _E
    ar = jnp.arange(npad, dtype=jnp.int32)
    srcb = jnp.concatenate([src, ar % 128]).reshape(_NW, _K, _C)
    dstb = jnp.concatenate([dst, _N + (ar % (_NPAD - _N))]).reshape(_NW, _K, _C)

    zeros = jnp.zeros((_NPAD, _W), jnp.float32)
    ones = jnp.ones((_C, _W), jnp.float32)

    dp = _sc_degree(dstb, zeros, ones)
    hs, dis = _tc_prep(x, W1, dp)

    for b_prev, W_next in ((b1, W2), (b2, W3), (b3, W4)):
        p = _sc_aggregate(hs, srcb, dstb, zeros)
        hs = _tc_mid(p, hs, dis, b_prev.reshape(1, _HID), W_next)

    p = _sc_aggregate(hs, srcb, dstb, zeros)
    return _tc_final(
        p, hs, dis, b4.reshape(1, _HID), batch.reshape(1, _N), Wl,
        bl.reshape(1, _OUT),
    )


# speedup vs baseline: 26.8585x; 1.1089x over previous
import functools

import jax
import jax.numpy as jnp
from jax import lax
from jax.experimental import pallas as pl
from jax.experimental.pallas import tpu as pltpu
from jax.experimental.pallas import tpu_sc as plsc

_N = 10000
_E = 320000
_IN = 128
_HID = 64
_OUT = 128
_G = 64
_W = 128

_NC = 2
_NS = 16
_NW = _NC * _NS
_C = 128
_K = 80
_EPAD = _NW * _K * _C
_NPAD = 10112
_RPT = _NPAD // _NS

_mesh = plsc.VectorSubcoreMesh(core_axis_name="c", subcore_axis_name="s")


@functools.partial(
    pl.kernel,
    out_type=jax.ShapeDtypeStruct((_NC, _NPAD), jnp.float32),
    mesh=_mesh,
    scratch_types=[
        pltpu.VMEM((_K, _C), jnp.int32),
        pltpu.VMEM((_C,), jnp.float32),
        pltpu.SemaphoreType.DMA,
        pltpu.VMEM_SHARED((_NPAD,), jnp.float32),
    ],
)
def _sc_degree(dstb, zeros, ones_hbm, out, didx, ones_v, ssem, acc):
    c = lax.axis_index("c")
    s = lax.axis_index("s")
    w = c * _NS + s
    r0 = s * 640

    @pl.when(s < 15)
    def _():
        pltpu.sync_copy(zeros.at[pl.ds(r0, 640)], acc.at[pl.ds(r0, 640)])

    @pl.when(s == 15)
    def _():
        pltpu.sync_copy(zeros.at[pl.ds(9600, 512)], acc.at[pl.ds(9600, 512)])

    pltpu.sync_copy(dstb.at[w], didx)
    pltpu.sync_copy(ones_hbm, ones_v)
    plsc.subcore_barrier()

    def group(gi, carry):
        base = gi * 8
        descs = [
            pltpu.async_copy(ones_v, acc.at[didx.at[base + j]], ssem, add=True)
            for j in range(8)
        ]
        for d in descs:
            d.wait()
        return carry

    lax.fori_loop(0, _K // 8, group, 0)
    plsc.subcore_barrier()

    @pl.when(s < 15)
    def _():
        pltpu.sync_copy(acc.at[pl.ds(r0, 640)], out.at[c, pl.ds(r0, 640)])

    @pl.when(s == 15)
    def _():
        pltpu.sync_copy(acc.at[pl.ds(9600, 512)], out.at[c, pl.ds(9600, 512)])


@functools.partial(
    pl.kernel,
    out_type=jax.ShapeDtypeStruct((_NC, _NPAD, _W), jnp.float32),
    mesh=_mesh,
    scratch_types=[
        pltpu.VMEM((_K, _C), jnp.int32),
        pltpu.VMEM((2, _C), jnp.int32),
        pltpu.VMEM((2, _C, _W), jnp.float32),
        pltpu.SemaphoreType.DMA,
        pltpu.SemaphoreType.DMA,
        pltpu.SemaphoreType.DMA,
        pltpu.SemaphoreType.DMA,
        pltpu.SemaphoreType.DMA,
        pltpu.SemaphoreType.DMA,
        pltpu.VMEM_SHARED((_NPAD, _W), jnp.float32),
    ],
)
def _sc_aggregate(hs, srcb, dstb, zeros, out, sidx, didx, rows, gsem0, gsem1,
                  ssem0, ssem1, isem0, isem1, acc):
    c = lax.axis_index("c")
    s = lax.axis_index("s")
    w = c * _NS + s
    r0 = s * _RPT

    pltpu.sync_copy(zeros.at[pl.ds(r0, _RPT)], acc.at[pl.ds(r0, _RPT)])
    pltpu.sync_copy(srcb.at[w], sidx)
    plsc.subcore_barrier()

    def g_start(k, buf, gsem, isem):
        pltpu.async_copy(dstb.at[w, k], didx.at[buf], isem)
        pltpu.async_copy(hs.at[sidx.at[k]], rows.at[buf], gsem)

    def g_wait(buf, gsem, isem):
        pltpu.make_async_copy(hs.at[sidx.at[0]], rows.at[buf], gsem).wait()
        pltpu.make_async_copy(dstb.at[w, 0], didx.at[buf], isem).wait()

    def s_start(buf, ssem):
        pltpu.async_copy(rows.at[buf], acc.at[didx.at[buf]], ssem, add=True)

    def s_wait(buf, ssem):
        pltpu.make_async_copy(rows.at[buf], acc.at[didx.at[buf]], ssem).wait()

    g_start(0, 0, gsem0, isem0)

    def step(i, carry):
        k0 = 2 * i
        pl.when(i >= 1)(lambda: s_wait(1, ssem1))
        g_start(k0 + 1, 1, gsem1, isem1)
        g_wait(0, gsem0, isem0)
        s_start(0, ssem0)

        @pl.when(i + 1 < _K // 2)
        def _():
            s_wait(0, ssem0)
            g_start(k0 + 2, 0, gsem0, isem0)

        g_wait(1, gsem1, isem1)
        s_start(1, ssem1)
        return carry

    lax.fori_loop(0, _K // 2, step, 0)
    s_wait(0, ssem0)
    s_wait(1, ssem1)
    plsc.subcore_barrier()
    pltpu.sync_copy(acc.at[pl.ds(r0, _RPT)], out.at[c, pl.ds(r0, _RPT)])


def _tc_prep_body(x_ref, w_ref, dp_ref, hs_ref, dis_ref):
    deg = dp_ref[0, : _N, 0:1] + dp_ref[1, : _N, 0:1] + 1.0
    dis = lax.rsqrt(deg)
    h = jnp.dot(x_ref[...], w_ref[...], preferred_element_type=jnp.float32)
    hs_ref[0:_N, 0:_HID] = h * dis
    hs_ref[0:_N, _HID:_W] = jnp.zeros((_N, _W - _HID), jnp.float32)
    hs_ref[_N:_NPAD, :] = jnp.zeros((_NPAD - _N, _W), jnp.float32)
    dis_ref[...] = dis


def _tc_mid_body(p_ref, hs_ref, dis_ref, b_ref, w_ref, o_ref):
    dis = dis_ref[...]
    agg = (p_ref[0, : _N, 0:_HID] + p_ref[1, : _N, 0:_HID]
           + hs_ref[0:_N, 0:_HID]) * dis
    a = jnp.maximum(agg + b_ref[...], 0.0)
    o_ref[0:_N, 0:_HID] = (
        jnp.dot(a, w_ref[...], preferred_element_type=jnp.float32) * dis
    )
    o_ref[0:_N, _HID:_W] = jnp.zeros((_N, _W - _HID), jnp.float32)
    o_ref[_N:_NPAD, :] = jnp.zeros((_NPAD - _N, _W), jnp.float32)


def _tc_final_body(p_ref, hs_ref, dis_ref, b_ref, bt_ref, wl_ref, bl_ref, o_ref):
    dis = dis_ref[...]
    agg = (p_ref[0, : _N, 0:_HID] + p_ref[1, : _N, 0:_HID]
           + hs_ref[0:_N, 0:_HID]) * dis
    a = jnp.maximum(agg + b_ref[...], 0.0)
    gids = lax.broadcasted_iota(jnp.int32, (_G, _N), 0)
    oh = (bt_ref[...] == gids).astype(jnp.float32)
    sums = jnp.dot(oh, a, preferred_element_type=jnp.float32)
    cnt = jnp.sum(oh, axis=1, keepdims=True)
    pooled = sums / jnp.maximum(cnt, 1.0)
    o_ref[...] = (
        jnp.dot(pooled, wl_ref[...], preferred_element_type=jnp.float32)
        + bl_ref[...]
    )


_tc_prep = pl.pallas_call(
    _tc_prep_body,
    out_shape=(
        jax.ShapeDtypeStruct((_NPAD, _W), jnp.float32),
        jax.ShapeDtypeStruct((_N, 1), jnp.float32),
    ),
)

_tc_mid = pl.pallas_call(
    _tc_mid_body,
    out_shape=jax.ShapeDtypeStruct((_NPAD, _W), jnp.float32),
)

_tc_final = pl.pallas_call(
    _tc_final_body,
    out_shape=jax.ShapeDtypeStruct((_G, _OUT), jnp.float32),
)


def kernel(x, edge_index, batch, W1, b1, W2, b2, W3, b3, W4, b4, Wl, bl):
    src = edge_index[0]
    dst = edge_index[1]
    npad = _EPAD - _E
    ar = jnp.arange(npad, dtype=jnp.int32)
    srcb = jnp.concatenate([src, ar % 128]).reshape(_NW, _K, _C)
    dstb = jnp.concatenate([dst, _N + (ar % (_NPAD - _N))]).reshape(_NW, _K, _C)

    zeros = jnp.zeros((_NPAD, _W), jnp.float32)
    zeros1 = jnp.zeros((_NPAD,), jnp.float32)
    ones1 = jnp.ones((_C,), jnp.float32)

    dp = _sc_degree(dstb, zeros1, ones1)
    hs, dis = _tc_prep(x, W1, dp.reshape(_NC, _NPAD, 1))

    for b_prev, W_next in ((b1, W2), (b2, W3), (b3, W4)):
        p = _sc_aggregate(hs, srcb, dstb, zeros)
        hs = _tc_mid(p, hs, dis, b_prev.reshape(1, _HID), W_next)

    p = _sc_aggregate(hs, srcb, dstb, zeros)
    return _tc_final(
        p, hs, dis, b4.reshape(1, _HID), batch.reshape(1, _N), Wl,
        bl.reshape(1, _OUT),
    )
